# baseline (speedup 1.0000x reference)
import functools

import jax
import jax.numpy as jnp
from jax import lax
from jax.experimental import pallas as pl
from jax.experimental.pallas import tpu as pltpu
from jax.experimental.pallas import tpu_sc as plsc

D = 64
NC, NS = 2, 16
NW = NC * NS
SPLIT = 128
GW = 4
NG = 8


def _make_gather(b, s):
    wrows = b // NW
    assert wrows == GW * NG
    mesh = plsc.VectorSubcoreMesh(core_axis_name="c", subcore_axis_name="s")

    @functools.partial(
        pl.kernel,
        mesh=mesh,
        compiler_params=pltpu.CompilerParams(use_tc_tiling_on_sc=False),
        out_type=jax.ShapeDtypeStruct((b, s, D), jnp.float32),
        scratch_types=[
            pltpu.VMEM((wrows, s), jnp.int32),
            pltpu.VMEM((GW, s, D), jnp.float32),
            pltpu.VMEM((GW, s, D), jnp.float32),
            pltpu.SemaphoreType.DMA,
            pltpu.SemaphoreType.DMA,
            pltpu.SemaphoreType.DMA,
            pltpu.SemaphoreType.DMA,
        ],
    )
    def gather_kernel(words_hbm, table_hbm, out_hbm,
                      idx_v, rows0, rows1, gsem0, gsem1, osem0, osem1):
        wid = lax.axis_index("s") * NC + lax.axis_index("c")
        row0 = wid * wrows
        pltpu.sync_copy(words_hbm.at[pl.ds(row0, wrows)], idx_v)

        def group_copies(g, rows, gsem, start):
            for k in range(GW):
                wr = g * GW + k
                for lo, n in ((0, SPLIT), (SPLIT, s - SPLIT)):
                    src = table_hbm.at[idx_v.at[wr, pl.ds(lo, n)]]
                    dst = rows.at[k, pl.ds(lo, n)]
                    if start:
                        pltpu.async_copy(src, dst, gsem)
                    else:
                        pltpu.make_async_copy(src, dst, gsem).wait()

        def out_start(g, rows, osem):
            pltpu.async_copy(rows, out_hbm.at[pl.ds(row0 + g * GW, GW)], osem)

        def out_wait(rows, osem):
            pltpu.make_async_copy(
                rows, out_hbm.at[pl.ds(row0, GW)], osem
            ).wait()

        group_copies(0, rows0, gsem0, start=True)
        group_copies(1, rows1, gsem1, start=True)

        def body(k, carry):
            g0 = 2 * k
            group_copies(g0, rows0, gsem0, start=False)
            out_start(g0, rows0, osem0)
            out_wait(rows0, osem0)
            group_copies(g0 + 2, rows0, gsem0, start=True)
            group_copies(g0 + 1, rows1, gsem1, start=False)
            out_start(g0 + 1, rows1, osem1)
            out_wait(rows1, osem1)
            group_copies(g0 + 3, rows1, gsem1, start=True)
            return carry

        lax.fori_loop(0, NG // 2 - 1, body, 0)

        g_last = NG - 2
        group_copies(g_last, rows0, gsem0, start=False)
        out_start(g_last, rows0, osem0)
        group_copies(g_last + 1, rows1, gsem1, start=False)
        out_start(g_last + 1, rows1, osem1)
        out_wait(rows0, osem0)
        out_wait(rows1, osem1)

    return gather_kernel


def kernel(words, table):
    b, s = words.shape
    return _make_gather(b, s)(words, table)


# speedup vs baseline: 1.0114x; 1.0114x over previous
import functools

import jax
import jax.numpy as jnp
from jax import lax
from jax.experimental import pallas as pl
from jax.experimental.pallas import tpu as pltpu
from jax.experimental.pallas import tpu_sc as plsc

D = 64
NC, NS = 2, 16
NW = NC * NS
GS = 25
NG = 8


def _make_gather(s, b):
    bw = b // NW
    assert s == GS * NG
    mesh = plsc.VectorSubcoreMesh(core_axis_name="c", subcore_axis_name="s")

    @functools.partial(
        pl.kernel,
        mesh=mesh,
        compiler_params=pltpu.CompilerParams(use_tc_tiling_on_sc=False),
        out_type=jax.ShapeDtypeStruct((s, b, D), jnp.float32),
        scratch_types=[
            pltpu.VMEM((s, bw), jnp.int32),
            pltpu.VMEM((GS, bw, D), jnp.float32),
            pltpu.VMEM((GS, bw, D), jnp.float32),
            pltpu.SemaphoreType.DMA,
            pltpu.SemaphoreType.DMA,
            pltpu.SemaphoreType.DMA,
            pltpu.SemaphoreType.DMA,
        ],
    )
    def gather_kernel(words_hbm, table_hbm, out_hbm,
                      idx_v, rows0, rows1, gsem0, gsem1, osem0, osem1):
        wid = lax.axis_index("s") * NC + lax.axis_index("c")
        col0 = wid * bw
        pltpu.sync_copy(words_hbm.at[:, pl.ds(col0, bw)], idx_v)

        def group_copies(g, rows, gsem, start):
            for k in range(GS):
                src = table_hbm.at[idx_v.at[g * GS + k]]
                dst = rows.at[k]
                if start:
                    pltpu.async_copy(src, dst, gsem)
                else:
                    pltpu.make_async_copy(src, dst, gsem).wait()

        def out_start(g, rows, osem):
            pltpu.async_copy(
                rows,
                out_hbm.at[pl.ds(g * GS, GS), pl.ds(col0, bw)],
                osem,
            )

        def out_wait(rows, osem):
            pltpu.make_async_copy(
                rows, out_hbm.at[pl.ds(0, GS), pl.ds(col0, bw)], osem
            ).wait()

        group_copies(0, rows0, gsem0, start=True)
        group_copies(1, rows1, gsem1, start=True)

        def body(k, carry):
            g0 = 2 * k
            group_copies(g0, rows0, gsem0, start=False)
            out_start(g0, rows0, osem0)
            out_wait(rows0, osem0)
            group_copies(g0 + 2, rows0, gsem0, start=True)
            group_copies(g0 + 1, rows1, gsem1, start=False)
            out_start(g0 + 1, rows1, osem1)
            out_wait(rows1, osem1)
            group_copies(g0 + 3, rows1, gsem1, start=True)
            return carry

        lax.fori_loop(0, NG // 2 - 1, body, 0)

        g_last = NG - 2
        group_copies(g_last, rows0, gsem0, start=False)
        out_start(g_last, rows0, osem0)
        group_copies(g_last + 1, rows1, gsem1, start=False)
        out_start(g_last + 1, rows1, osem1)
        out_wait(rows0, osem0)
        out_wait(rows1, osem1)

    return gather_kernel


def kernel(words, table):
    b, s = words.shape
    out_t = _make_gather(s, b)(words.T, table)
    return out_t.transpose(1, 0, 2)
